# baseline (speedup 1.0000x reference)
import math

import jax
import jax.numpy as jnp
from jax.experimental import pallas as pl
from jax.experimental.pallas import tpu as pltpu

B, C, H, W = 16, 6, 500, 256
LD = 4
HB, WB = 10, 32
ROWS = HB * WB
L2PI = float(math.log(2.0 * math.pi))
F32 = jnp.float32


def _iota2(shape):
    r = jax.lax.broadcasted_iota(jnp.int32, shape, 0)
    c = jax.lax.broadcasted_iota(jnp.int32, shape, 1)
    return r, c


def _gather_mat(win, kw, phase, wout, valid):
    r, c = _iota2((win, wout))
    return ((c < valid) & (r == kw * c + phase)).astype(F32)


def _scatter_mat(win, kw, phase, wout, valid_in):
    r, c = _iota2((win, wout))
    return ((r < valid_in) & (c == kw * r + phase)).astype(F32)


def _conv_static(src, w, b, p0, rows, wd, act):
    cin = src.shape[0]
    cout = w.shape[3]
    y = jnp.zeros((cout, rows, wd), F32) + b.reshape(cout, 1, 1)
    for ci in range(cin):
        slab = src[ci, p0 - 1:p0 + rows + 1, 0:wd + 2]
        for dy in range(3):
            for dx in range(3):
                y = y + slab[dy:dy + rows, dx:dx + wd][None] * w[dy, dx, ci][:, None, None]
    if act:
        y = jnp.maximum(y, 0.01 * y)
    return y


def _zero_border(ref):
    c0, hp, wp = ref.shape
    ref[:, 0:1, :] = jnp.zeros((c0, 1, wp), F32)
    ref[:, hp - 1:hp, :] = jnp.zeros((c0, 1, wp), F32)
    ref[:, :, 0:1] = jnp.zeros((c0, hp, 1), F32)
    ref[:, :, wp - 1:wp] = jnp.zeros((c0, hp, 1), F32)


def _pool_chunk(y, kh, gmats):
    cout, rows, ww = y.shape
    y2 = y.reshape(cout, rows // kh, kh * ww)
    m = y2[:, :, 0:ww]
    for k in range(1, kh):
        m = jnp.maximum(m, y2[:, :, k * ww:(k + 1) * ww])
    m2 = m.reshape(cout * (rows // kh), ww)
    z = jnp.maximum(m2 @ gmats[0], m2 @ gmats[1])
    return z.reshape(cout, rows // kh, gmats[0].shape[1])


def _deconv_chunk(x, w, b, smats, wout):
    kh, kw, cin, cout = w.shape
    rows, ww = x.shape[1], x.shape[2]
    parts = []
    for i in range(kh):
        row = None
        for j in range(kw):
            acc = jnp.zeros((cout, rows, ww), F32) + b.reshape(cout, 1, 1)
            for ci in range(cin):
                acc = acc + x[ci][None] * w[kh - 1 - i, kw - 1 - j, ci][:, None, None]
            term = acc.reshape(cout * rows, ww) @ smats[j]
            row = term if row is None else row + term
        parts.append(row.reshape(cout, rows, wout))
    t = jnp.concatenate(parts, axis=2)
    return t.reshape(cout, kh * rows, wout)


def _enc_kernel(xp_ref, c1w_r, c1b_r, c2w_r, c2b_r, c3w_r, c3b_r, feats_ref,
                h1p, h2p, h3r):
    c1w, c1b = c1w_r[...], c1b_r[...]
    c2w, c2b = c2w_r[...], c2b_r[...]
    c3w, c3b = c3w_r[...], c3b_r[...]
    _zero_border(h1p)
    _zero_border(h2p)

    g1 = (_gather_mat(256, 2, 0, 128, 128), _gather_mat(256, 2, 1, 128, 128))
    g2 = (_gather_mat(128, 2, 0, 128, 64), _gather_mat(128, 2, 1, 128, 64))
    g3 = (_gather_mat(128, 2, 0, 128, 32), _gather_mat(128, 2, 1, 128, 32))

    for i in range(25):
        y = _conv_static(xp_ref.at[0], c1w, c1b, 1 + 20 * i, 20, 256, act=True)
        h1p[:, 1 + 10 * i:11 + 10 * i, 1:129] = _pool_chunk(y, 2, g1)

    for i in range(10):
        y = _conv_static(h1p, c2w, c2b, 1 + 25 * i, 25, 128, act=True)
        h2p[:, 1 + 5 * i:6 + 5 * i, 1:129] = _pool_chunk(y, 5, g2)

    for i in range(5):
        y = _conv_static(h2p, c3w, c3b, 1 + 10 * i, 10, 128, act=True)
        h3r[:, 2 * i:2 * i + 2, :] = _pool_chunk(y, 5, g3)

    h3 = h3r[...]
    for q in range(HB):
        feats_ref[0, :, q * WB:(q + 1) * WB] = h3[:, q, 0:WB]


def _dec_kernel_a(feats_ref, eps_ref,
                  muw, mub, lvw, lvb, fW1, fb1, fW2, fb2, fW3, fb3,
                  linw, linb, d1w_r, d1b_r, c4w_r, c4b_r, d2w_r, d2b_r,
                  g2_ref, mu_ref, lv_ref, kld_ref, z_ref,
                  sq1):
    feats = feats_ref[0]
    mu = muw[...] @ feats + mub[...]
    lv = lvw[...] @ feats + lvb[...]
    std = jnp.exp(0.5 * lv)
    z0 = mu + eps_ref[0] * std
    z = z0
    log_det = jnp.zeros((1, ROWS), F32)
    for i in range(10):
        if i % 2 == 0:
            a, bb = z[:2, :], z[2:, :]
        else:
            bb, a = z[:2, :], z[2:, :]
        hm = jnp.tanh(fW1[i] @ a + fb1[i])
        hm = jnp.tanh(fW2[i] @ hm + fb2[i])
        st = fW3[i] @ hm + fb3[i]
        s = jnp.tanh(st[:2, :])
        t = st[2:, :]
        bnew = bb * jnp.exp(s) + t
        log_det = log_det + jnp.sum(s, axis=0, keepdims=True)
        z = (jnp.concatenate([a, bnew], axis=0) if i % 2 == 0
             else jnp.concatenate([bnew, a], axis=0))
    log_q0 = jnp.sum(-0.5 * ((z0 - mu) / std) ** 2 - 0.5 * lv - 0.5 * L2PI,
                     axis=0, keepdims=True)
    log_p = jnp.sum(-0.5 * z * z - 0.5 * L2PI, axis=0, keepdims=True)
    mu_ref[...] = mu.T
    lv_ref[...] = lv.T
    z_ref[...] = z.T
    kld_ref[...] = jnp.sum(log_q0 - log_det - log_p).reshape(1, 1, 1)

    d1w, d1b = d1w_r[...], d1b_r[...]
    c4w, c4b = c4w_r[...], c4b_r[...]
    d2w, d2b = d2w_r[...], d2b_r[...]
    sq2 = g2_ref.at[0]
    _zero_border(sq1)
    _zero_border(sq2)

    dec = linw[...] @ z + linb[...]
    zpad = jnp.zeros((32, 1, 128 - WB), F32)
    dplanes = jnp.concatenate(
        [jnp.concatenate([dec[:, q * WB:(q + 1) * WB][:, None, :], zpad],
                         axis=2) for q in range(HB)], axis=1)

    sm1 = (_scatter_mat(128, 2, 0, 128, 32), _scatter_mat(128, 2, 1, 128, 32))
    sm2 = (_scatter_mat(128, 2, 0, 128, 64), _scatter_mat(128, 2, 1, 128, 64))

    for i in range(5):
        t = _deconv_chunk(dplanes[:, i * 2:(i + 1) * 2, :], d1w, d1b, sm1, 128)
        sq1[:, 1 + 10 * i:11 + 10 * i, 1:129] = t

    for i in range(10):
        g = _conv_static(sq1, c4w, c4b, 1 + 5 * i, 5, 128, act=False)
        t = _deconv_chunk(g, d2w, d2b, sm2, 128)
        sq2[:, 1 + 25 * i:26 + 25 * i, 1:129] = t


def _dec_kernel_b(g2_ref, c5w_r, c5b_r, d3w_r, d3b_r, c6w_r, c6b_r,
                  out_ref, sq3):
    c5w, c5b = c5w_r[...], c5b_r[...]
    d3w, d3b = d3w_r[...], d3b_r[...]
    c6w, c6b = c6w_r[...], c6b_r[...]
    _zero_border(sq3)
    sq2 = g2_ref.at[0]
    sm3 = (_scatter_mat(128, 2, 0, 256, 128), _scatter_mat(128, 2, 1, 256, 128))

    for i in range(25):
        g = _conv_static(sq2, c5w, c5b, 1 + 10 * i, 10, 128, act=False)
        t = _deconv_chunk(g, d3w, d3b, sm3, 256)
        sq3[:, 1 + 20 * i:21 + 20 * i, 1:257] = t

    for i in range(20):
        y = _conv_static(sq3, c6w, c6b, 1 + 25 * i, 25, 256, act=False)
        out_ref[0, :, 25 * i:25 * i + 25, :] = y


def kernel(x, params, eps):
    p = params
    xpad = jnp.pad(x, ((0, 0), (0, 0), (1, 1), (1, 1)))
    full = pl.BlockSpec()
    row = lambda c: pl.BlockSpec((ROWS, c), lambda b: (b, 0))
    feats = pl.pallas_call(
        _enc_kernel,
        grid=(B,),
        in_specs=[pl.BlockSpec((1, C, H + 2, W + 2), lambda b: (b, 0, 0, 0))]
                 + [full] * 6,
        out_specs=pl.BlockSpec((1, 32, ROWS), lambda b: (b, 0, 0)),
        out_shape=jax.ShapeDtypeStruct((B, 32, ROWS), F32),
        scratch_shapes=[pltpu.VMEM((8, 252, 130), F32),
                        pltpu.VMEM((16, 52, 130), F32),
                        pltpu.VMEM((32, 10, 128), F32)],
        compiler_params=pltpu.CompilerParams(vmem_limit_bytes=100 * 1024 * 1024),
    )(xpad, p['c1w'], p['c1b'][None, :], p['c2w'], p['c2b'][None, :],
      p['c3w'], p['c3b'][None, :])
    g2, mu, lv, kld, z = pl.pallas_call(
        _dec_kernel_a,
        grid=(B,),
        in_specs=[pl.BlockSpec((1, 32, ROWS), lambda b: (b, 0, 0)),
                  pl.BlockSpec((1, LD, ROWS), lambda b: (b, 0, 0))] + [full] * 18,
        out_specs=[pl.BlockSpec((1, 16, 252, 130), lambda b: (b, 0, 0, 0)),
                   row(LD), row(LD), pl.BlockSpec((1, 1, 1), lambda b: (b, 0, 0)),
                   row(LD)],
        out_shape=[jax.ShapeDtypeStruct((B, 16, 252, 130), F32),
                   jax.ShapeDtypeStruct((B * ROWS, LD), F32),
                   jax.ShapeDtypeStruct((B * ROWS, LD), F32),
                   jax.ShapeDtypeStruct((B, 1, 1), F32),
                   jax.ShapeDtypeStruct((B * ROWS, LD), F32)],
        scratch_shapes=[pltpu.VMEM((32, 52, 130), F32)],
        compiler_params=pltpu.CompilerParams(vmem_limit_bytes=100 * 1024 * 1024),
    )(feats, eps.reshape(B, ROWS, LD).transpose(0, 2, 1),
      p['muw'].T, p['mub'][:, None], p['lvw'].T, p['lvb'][:, None],
      jnp.transpose(p['fW1'], (0, 2, 1)), p['fb1'][:, :, None],
      jnp.transpose(p['fW2'], (0, 2, 1)), p['fb2'][:, :, None],
      jnp.transpose(p['fW3'], (0, 2, 1)), p['fb3'][:, :, None],
      p['linw'].T, p['linb'][:, None],
      p['d1w'], p['d1b'][None, :], p['c4w'], p['c4b'][None, :],
      p['d2w'], p['d2b'][None, :])
    out = pl.pallas_call(
        _dec_kernel_b,
        grid=(B,),
        in_specs=[pl.BlockSpec((1, 16, 252, 130), lambda b: (b, 0, 0, 0))]
                 + [full] * 6,
        out_specs=pl.BlockSpec((1, C, H, W), lambda b: (b, 0, 0, 0)),
        out_shape=jax.ShapeDtypeStruct((B, C, H, W), F32),
        scratch_shapes=[pltpu.VMEM((8, 502, 258), F32)],
        compiler_params=pltpu.CompilerParams(vmem_limit_bytes=100 * 1024 * 1024),
    )(g2, p['c5w'], p['c5b'][None, :], p['d3w'], p['d3b'][None, :],
      p['c6w'], p['c6b'][None, :])
    return (out, mu, lv, kld.reshape(B), z)
